# Initial kernel scaffold; baseline (speedup 1.0000x reference)
import jax
import jax.numpy as jnp
from jax.experimental import pallas as pl


def kernel(x, edge_index, W1, b1, W2, b2, W3, b3, Wl, bl):
    raise NotImplementedError("write your pallas kernel here")


# speedup vs baseline: 2.0952x; 2.0952x over previous
import jax
import jax.numpy as jnp
from jax import lax
from jax.experimental import pallas as pl
from jax.experimental.pallas import tpu as pltpu
from jax.experimental.pallas import tpu_sc as plsc

N = 100000
D = 32
H = 16
E = 1600000

NC = 2
NS = 16

NP = 102400
RNG = NP // NS
ACC = RNG * H
ACCP = 102528
TRASH = ACC

CAPR = 12516
BN = 2048
GRID = NP // BN

_MESH = plsc.VectorSubcoreMesh(
    core_axis_name="c", subcore_axis_name="s", num_cores=NC, num_subcores=NS
)

_f32 = jnp.float32
_i32 = jnp.int32


def _lane_scalar(vref, lane):
    return vref[pl.ds(lane * 16, 16)][0]


def _deg_body(idx16, offs_h, zz, deg1, acc, idxc, offv):
    c = lax.axis_index("c")
    s = lax.axis_index("s")

    @pl.when(c == 0)
    def _():
        pltpu.sync_copy(zz, acc)
        pltpu.sync_copy(offs_h, offv)
        off = _lane_scalar(offv, s)
        cnt = _lane_scalar(offv, 16 + s)
        ones16 = jnp.full((16,), 1.0, _f32)

        def erow(k, _):
            row = off + k
            pltpu.sync_copy(idx16.at[pl.ds(row * 2048, 2048)], idxc)

            def edge(j, _):
                iv = idxc[pl.ds(j * 16, 16)]
                plsc.addupdate_scatter(acc, [iv], ones16)
                return 0

            lax.fori_loop(0, 128, edge, 0)
            return 0

        lax.fori_loop(0, cnt, erow, 0)
        pltpu.sync_copy(acc.at[pl.ds(0, ACC)], deg1.at[pl.ds(s * ACC, ACC)])


_deg_call = pl.kernel(
    _deg_body,
    out_type=jax.ShapeDtypeStruct((NP * H,), _f32),
    mesh=_MESH,
    scratch_types=[
        pltpu.VMEM((ACCP,), _f32),
        pltpu.VMEM((2048,), _i32),
        pltpu.VMEM((512,), _i32),
    ],
    compiler_params=pltpu.CompilerParams(needs_layout_passes=False),
)


def _agg_body(uwa, uwb, src1, idx16, offs_h, zz, ga1, gb1,
              acc, idxv, idxc, rows, offv, sem):
    c = lax.axis_index("c")
    s = lax.axis_index("s")

    def run(u_hbm, g_hbm):
        pltpu.sync_copy(zz, acc)
        pltpu.sync_copy(offs_h, offv)
        off = _lane_scalar(offv, s)
        cnt = _lane_scalar(offv, 16 + s)

        def erow(k, _):
            row = off + k
            pltpu.sync_copy(src1.at[pl.ds(row * 128, 128)], idxv)
            pltpu.sync_copy(idx16.at[pl.ds(row * 2048, 2048)], idxc)
            pltpu.async_copy(u_hbm.at[idxv], rows, sem).wait()

            def edge(j, _):
                v = rows[j, pl.ds(0, 16)]
                iv = idxc[pl.ds(j * 16, 16)]
                plsc.addupdate_scatter(acc, [iv], v)
                return 0

            lax.fori_loop(0, 128, edge, 0)
            return 0

        lax.fori_loop(0, cnt, erow, 0)
        pltpu.sync_copy(acc.at[pl.ds(0, ACC)], g_hbm.at[pl.ds(s * ACC, ACC)])

    @pl.when(c == 0)
    def _():
        run(uwa, ga1)

    @pl.when(c == 1)
    def _():
        run(uwb, gb1)


_agg_call = pl.kernel(
    _agg_body,
    out_type=(
        jax.ShapeDtypeStruct((NP * H,), _f32),
        jax.ShapeDtypeStruct((NP * H,), _f32),
    ),
    mesh=_MESH,
    scratch_types=[
        pltpu.VMEM((ACCP,), _f32),
        pltpu.VMEM((128,), _i32),
        pltpu.VMEM((2048,), _i32),
        pltpu.VMEM((128, 128), _f32),
        pltpu.VMEM((512,), _i32),
        pltpu.SemaphoreType.DMA,
    ],
    compiler_params=pltpu.CompilerParams(needs_layout_passes=False),
)


def _dinv_col(deg_ref):
    return lax.rsqrt(deg_ref[...][:, :1] + 1.0)


def _widen(u):
    z = jnp.zeros((u.shape[0], 128 - H), _f32)
    return (
        jnp.concatenate([u[:, :H], z], axis=1),
        jnp.concatenate([u[:, H:], z], axis=1),
    )


def _tc_pre_body(x_ref, w_ref, deg_ref, oa_ref, ob_ref):
    dinv = _dinv_col(deg_ref)
    u = jnp.dot(x_ref[...], w_ref[...], preferred_element_type=_f32) * dinv
    oa_ref[...], ob_ref[...] = _widen(u)


def _tc_mid_body(ga_ref, gb_ref, ua_ref, ub_ref, deg_ref, w_ref, b_ref,
                 oa_ref, ob_ref):
    dinv = _dinv_col(deg_ref)
    gm = jnp.concatenate(
        [ga_ref[...] + ua_ref[...][:, :H], gb_ref[...] + ub_ref[...][:, :H]],
        axis=1,
    )
    h = jnp.maximum(gm * dinv + b_ref[...], 0.0)
    u = jnp.dot(h, w_ref[...], preferred_element_type=_f32) * dinv
    oa_ref[...], ob_ref[...] = _widen(u)


def _tc_fin_body(ga_ref, gb_ref, ua_ref, ub_ref, deg_ref, b_ref, wl_ref,
                 bl_ref, o_ref):
    dinv = _dinv_col(deg_ref)
    gm = jnp.concatenate(
        [ga_ref[...] + ua_ref[...][:, :H], gb_ref[...] + ub_ref[...][:, :H]],
        axis=1,
    )
    h = gm * dinv + b_ref[...]
    o_ref[...] = jnp.dot(h, wl_ref[...], preferred_element_type=_f32) + bl_ref[...]


_h_spec = pl.BlockSpec((BN, H), lambda i: (i, 0))
_w_spec = pl.BlockSpec((BN, 128), lambda i: (i, 0))
_full_spec = lambda shape: pl.BlockSpec(shape, lambda i: tuple(0 for _ in shape))
_w_out = [jax.ShapeDtypeStruct((NP, 128), _f32)] * 2

_tc_pre = pl.pallas_call(
    _tc_pre_body,
    grid=(GRID,),
    in_specs=[pl.BlockSpec((BN, D), lambda i: (i, 0)), _full_spec((D, D)), _h_spec],
    out_specs=[_w_spec] * 2,
    out_shape=_w_out,
)

_tc_mid = pl.pallas_call(
    _tc_mid_body,
    grid=(GRID,),
    in_specs=[_h_spec, _h_spec, _w_spec, _w_spec, _h_spec,
              _full_spec((D, D)), _full_spec((1, D))],
    out_specs=[_w_spec] * 2,
    out_shape=_w_out,
)

_tc_fin = pl.pallas_call(
    _tc_fin_body,
    grid=(GRID,),
    in_specs=[_h_spec, _h_spec, _w_spec, _w_spec, _h_spec,
              _full_spec((1, D)), _full_spec((D, 2)), _full_spec((1, 2))],
    out_specs=pl.BlockSpec((BN, 2), lambda i: (i, 0)),
    out_shape=jax.ShapeDtypeStruct((NP, 2), _f32),
)


def kernel(x, edge_index, W1, b1, W2, b2, W3, b3, Wl, bl):
    src = edge_index[0].astype(_i32)
    dst = edge_index[1].astype(_i32)

    order = jnp.argsort(dst)
    ss = src[order]
    ds_ = dst[order]
    bounds = jnp.searchsorted(ds_, (jnp.arange(17, dtype=_i32) * RNG)).astype(_i32)
    cnt = bounds[1:] - bounds[:-1]
    rows_t = (cnt + 127) // 128
    offr = jnp.concatenate([jnp.zeros((1,), _i32), jnp.cumsum(rows_t)[:-1]])
    b_e = ds_ // RNG
    pos = offr[b_e] * 128 + (jnp.arange(E, dtype=_i32) - bounds[b_e])
    src_p = jnp.zeros((CAPR * 128,), _i32).at[pos].set(ss)
    idx_p = jnp.full((CAPR * 128,), TRASH, _i32).at[pos].set((ds_ % RNG) * H)
    idx16 = (idx_p[:, None] + jnp.arange(H, dtype=_i32)[None, :]).reshape(-1)
    lanes = jnp.arange(16, dtype=_i32) * 16
    offs_h = (jnp.zeros((512,), _i32).at[lanes].set(offr)
              .at[256 + lanes].set(rows_t))
    zz = jnp.zeros((ACCP,), _f32)

    xp = jnp.concatenate([x, jnp.zeros((NP - N, D), x.dtype)], axis=0)

    deg = _deg_call(idx16, offs_h, zz).reshape(NP, H)

    uwa, uwb = _tc_pre(xp, W1, deg)
    ga1, gb1 = _agg_call(uwa, uwb, src_p, idx16, offs_h, zz)
    uwa, uwb = _tc_mid(ga1.reshape(NP, H), gb1.reshape(NP, H), uwa, uwb,
                       deg, W2, b1.reshape(1, D))
    ga1, gb1 = _agg_call(uwa, uwb, src_p, idx16, offs_h, zz)
    uwa, uwb = _tc_mid(ga1.reshape(NP, H), gb1.reshape(NP, H), uwa, uwb,
                       deg, W3, b2.reshape(1, D))
    ga1, gb1 = _agg_call(uwa, uwb, src_p, idx16, offs_h, zz)
    out = _tc_fin(ga1.reshape(NP, H), gb1.reshape(NP, H), uwa, uwb,
                  deg, b3.reshape(1, D), Wl, bl.reshape(1, 2))
    return out[:N]


# speedup vs baseline: 2.3355x; 1.1147x over previous
import jax
import jax.numpy as jnp
from jax import lax
from jax.experimental import pallas as pl
from jax.experimental.pallas import tpu as pltpu
from jax.experimental.pallas import tpu_sc as plsc

N = 100000
D = 32
H = 16
E = 1600000

NC = 2
NS = 16

NP = 102400
RNG = NP // NS
ACC = RNG * H
ACCP = 102528
TRASH = ACC

CAPR = 12516
BN = 2048
GRID = NP // BN

_MESH = plsc.VectorSubcoreMesh(
    core_axis_name="c", subcore_axis_name="s", num_cores=NC, num_subcores=NS
)

_f32 = jnp.float32
_i32 = jnp.int32


def _lane_scalar(vref, lane):
    return vref[pl.ds(lane * 16, 16)][0]


def _deg_body(idx16, offs_h, zz, deg1, acc, idxc, offv):
    c = lax.axis_index("c")
    s = lax.axis_index("s")

    @pl.when(c == 0)
    def _():
        pltpu.sync_copy(zz, acc)
        pltpu.sync_copy(offs_h, offv)
        off = _lane_scalar(offv, s)
        cnt = _lane_scalar(offv, 16 + s)
        ones16 = jnp.full((16,), 1.0, _f32)

        def erow(k, _):
            row = off + k
            pltpu.sync_copy(idx16.at[pl.ds(row * 2048, 2048)], idxc)

            def edge(j, _):
                iv = idxc[pl.ds(j * 16, 16)]
                plsc.addupdate_scatter(acc, [iv], ones16)
                return 0

            lax.fori_loop(0, 128, edge, 0)
            return 0

        lax.fori_loop(0, cnt, erow, 0)
        pltpu.sync_copy(acc.at[pl.ds(0, ACC)], deg1.at[pl.ds(s * ACC, ACC)])


_deg_call = pl.kernel(
    _deg_body,
    out_type=jax.ShapeDtypeStruct((NP * H,), _f32),
    mesh=_MESH,
    scratch_types=[
        pltpu.VMEM((ACCP,), _f32),
        pltpu.VMEM((2048,), _i32),
        pltpu.VMEM((512,), _i32),
    ],
    compiler_params=pltpu.CompilerParams(needs_layout_passes=False),
)


def _agg_body(uwa, uwb, src1, idx16, offs_h, zz, ga1, gb1,
              acc, idxv, idxc, rows, offv, sem, semi):
    c = lax.axis_index("c")
    s = lax.axis_index("s")

    def run(u_hbm, g_hbm):
        pltpu.sync_copy(zz, acc)
        pltpu.sync_copy(offs_h, offv)
        off = _lane_scalar(offv, s)
        cnt = _lane_scalar(offv, 16 + s)

        def fetch(row, p):
            pltpu.async_copy(
                src1.at[pl.ds(row * 128, 128)], idxv.at[pl.ds(p * 128, 128)], semi
            )
            pltpu.async_copy(
                idx16.at[pl.ds(row * 2048, 2048)],
                idxc.at[pl.ds(p * 2048, 2048)], semi,
            )

        @pl.when(cnt > 0)
        def _():
            fetch(off, 0)

        def erow(k, _):
            p = k & 1
            row = off + k
            pltpu.make_async_copy(
                src1.at[pl.ds(row * 128, 128)], idxv.at[pl.ds(p * 128, 128)], semi
            ).wait()
            pltpu.make_async_copy(
                idx16.at[pl.ds(row * 2048, 2048)],
                idxc.at[pl.ds(p * 2048, 2048)], semi,
            ).wait()

            @pl.when(k + 1 < cnt)
            def _():
                fetch(row + 1, (k + 1) & 1)

            pltpu.async_copy(
                u_hbm.at[idxv.at[pl.ds(p * 128, 128)]], rows, sem
            ).wait()
            base = p * 2048

            def edge(j, _):
                v = rows[j, pl.ds(0, 16)]
                iv = idxc[pl.ds(base + j * 16, 16)]
                plsc.addupdate_scatter(acc, [iv], v)
                return 0

            lax.fori_loop(0, 128, edge, 0)
            return 0

        lax.fori_loop(0, cnt, erow, 0)
        pltpu.sync_copy(acc.at[pl.ds(0, ACC)], g_hbm.at[pl.ds(s * ACC, ACC)])

    @pl.when(c == 0)
    def _():
        run(uwa, ga1)

    @pl.when(c == 1)
    def _():
        run(uwb, gb1)


_agg_call = pl.kernel(
    _agg_body,
    out_type=(
        jax.ShapeDtypeStruct((NP * H,), _f32),
        jax.ShapeDtypeStruct((NP * H,), _f32),
    ),
    mesh=_MESH,
    scratch_types=[
        pltpu.VMEM((ACCP,), _f32),
        pltpu.VMEM((256,), _i32),
        pltpu.VMEM((4096,), _i32),
        pltpu.VMEM((128, 128), _f32),
        pltpu.VMEM((512,), _i32),
        pltpu.SemaphoreType.DMA,
        pltpu.SemaphoreType.DMA,
    ],
    compiler_params=pltpu.CompilerParams(needs_layout_passes=False),
)


def _dinv_col(deg_ref):
    return lax.rsqrt(deg_ref[...][:, :1] + 1.0)


def _widen(u):
    z = jnp.zeros((u.shape[0], 128 - H), _f32)
    return (
        jnp.concatenate([u[:, :H], z], axis=1),
        jnp.concatenate([u[:, H:], z], axis=1),
    )


def _tc_pre_body(x_ref, w_ref, deg_ref, oa_ref, ob_ref):
    dinv = _dinv_col(deg_ref)
    u = jnp.dot(x_ref[...], w_ref[...], preferred_element_type=_f32) * dinv
    oa_ref[...], ob_ref[...] = _widen(u)


def _tc_mid_body(ga_ref, gb_ref, ua_ref, ub_ref, deg_ref, w_ref, b_ref,
                 oa_ref, ob_ref):
    dinv = _dinv_col(deg_ref)
    gm = jnp.concatenate(
        [ga_ref[...] + ua_ref[...][:, :H], gb_ref[...] + ub_ref[...][:, :H]],
        axis=1,
    )
    h = jnp.maximum(gm * dinv + b_ref[...], 0.0)
    u = jnp.dot(h, w_ref[...], preferred_element_type=_f32) * dinv
    oa_ref[...], ob_ref[...] = _widen(u)


def _tc_fin_body(ga_ref, gb_ref, ua_ref, ub_ref, deg_ref, b_ref, wl_ref,
                 bl_ref, o_ref):
    dinv = _dinv_col(deg_ref)
    gm = jnp.concatenate(
        [ga_ref[...] + ua_ref[...][:, :H], gb_ref[...] + ub_ref[...][:, :H]],
        axis=1,
    )
    h = gm * dinv + b_ref[...]
    o_ref[...] = jnp.dot(h, wl_ref[...], preferred_element_type=_f32) + bl_ref[...]


_h_spec = pl.BlockSpec((BN, H), lambda i: (i, 0))
_w_spec = pl.BlockSpec((BN, 128), lambda i: (i, 0))
_full_spec = lambda shape: pl.BlockSpec(shape, lambda i: tuple(0 for _ in shape))
_w_out = [jax.ShapeDtypeStruct((NP, 128), _f32)] * 2

_tc_pre = pl.pallas_call(
    _tc_pre_body,
    grid=(GRID,),
    in_specs=[pl.BlockSpec((BN, D), lambda i: (i, 0)), _full_spec((D, D)), _h_spec],
    out_specs=[_w_spec] * 2,
    out_shape=_w_out,
)

_tc_mid = pl.pallas_call(
    _tc_mid_body,
    grid=(GRID,),
    in_specs=[_h_spec, _h_spec, _w_spec, _w_spec, _h_spec,
              _full_spec((D, D)), _full_spec((1, D))],
    out_specs=[_w_spec] * 2,
    out_shape=_w_out,
)

_tc_fin = pl.pallas_call(
    _tc_fin_body,
    grid=(GRID,),
    in_specs=[_h_spec, _h_spec, _w_spec, _w_spec, _h_spec,
              _full_spec((1, D)), _full_spec((D, 2)), _full_spec((1, 2))],
    out_specs=pl.BlockSpec((BN, 2), lambda i: (i, 0)),
    out_shape=jax.ShapeDtypeStruct((NP, 2), _f32),
)


def kernel(x, edge_index, W1, b1, W2, b2, W3, b3, Wl, bl):
    src = edge_index[0].astype(_i32)
    dst = edge_index[1].astype(_i32)

    order = jnp.argsort(dst)
    ss = src[order]
    ds_ = dst[order]
    bounds = jnp.searchsorted(ds_, (jnp.arange(17, dtype=_i32) * RNG)).astype(_i32)
    cnt = bounds[1:] - bounds[:-1]
    rows_t = (cnt + 127) // 128
    offr = jnp.concatenate([jnp.zeros((1,), _i32), jnp.cumsum(rows_t)[:-1]])
    b_e = ds_ // RNG
    pos = offr[b_e] * 128 + (jnp.arange(E, dtype=_i32) - bounds[b_e])
    src_p = jnp.zeros((CAPR * 128,), _i32).at[pos].set(ss)
    idx_p = jnp.full((CAPR * 128,), TRASH, _i32).at[pos].set((ds_ % RNG) * H)
    idx16 = (idx_p[:, None] + jnp.arange(H, dtype=_i32)[None, :]).reshape(-1)
    lanes = jnp.arange(16, dtype=_i32) * 16
    offs_h = (jnp.zeros((512,), _i32).at[lanes].set(offr)
              .at[256 + lanes].set(rows_t))
    zz = jnp.zeros((ACCP,), _f32)

    xp = jnp.concatenate([x, jnp.zeros((NP - N, D), x.dtype)], axis=0)

    deg = _deg_call(idx16, offs_h, zz).reshape(NP, H)

    uwa, uwb = _tc_pre(xp, W1, deg)
    ga1, gb1 = _agg_call(uwa, uwb, src_p, idx16, offs_h, zz)
    uwa, uwb = _tc_mid(ga1.reshape(NP, H), gb1.reshape(NP, H), uwa, uwb,
                       deg, W2, b1.reshape(1, D))
    ga1, gb1 = _agg_call(uwa, uwb, src_p, idx16, offs_h, zz)
    uwa, uwb = _tc_mid(ga1.reshape(NP, H), gb1.reshape(NP, H), uwa, uwb,
                       deg, W3, b2.reshape(1, D))
    ga1, gb1 = _agg_call(uwa, uwb, src_p, idx16, offs_h, zz)
    out = _tc_fin(ga1.reshape(NP, H), gb1.reshape(NP, H), uwa, uwb,
                  deg, b3.reshape(1, D), Wl, bl.reshape(1, 2))
    return out[:N]
